# baseline (speedup 1.0000x reference)
import functools

import jax
import jax.numpy as jnp
import numpy as np
from jax import lax
from jax.experimental import pallas as pl
from jax.experimental.pallas import tpu as pltpu
from jax.experimental.pallas import tpu_sc as plsc

_P = 0.1
_MEAN = 0.0
_SIGMA = 0.01
_N = 524288
_D = 64
_PAIRS = _N
_PD = 2 * _D
_NW = 32
_BLK = 256
_NBLK_W = _PAIRS // (_BLK * _NW)
_CK = 128


def _rotl(x, d):
    return ((x << np.uint32(d)) | (x >> np.uint32(32 - d))).astype(np.uint32)


def _threefry2x32_pair(key, x0, x1):
    x = [x0.astype(np.uint32).copy(), x1.astype(np.uint32).copy()]
    rotations = [(13, 15, 26, 6), (17, 29, 16, 24)]
    ks = [key[0], key[1], np.uint32(key[0] ^ key[1] ^ np.uint32(0x1BD11BDA))]
    x[0] = (x[0] + ks[0]).astype(np.uint32)
    x[1] = (x[1] + ks[1]).astype(np.uint32)
    for i in range(5):
        for r in rotations[i % 2]:
            x[0] = (x[0] + x[1]).astype(np.uint32)
            x[1] = _rotl(x[1], r)
            x[1] = x[1] ^ x[0]
        x[0] = (x[0] + ks[(i + 1) % 3]).astype(np.uint32)
        x[1] = (x[1] + ks[(i + 2) % 3] + np.uint32(i + 1)).astype(np.uint32)
    return x[0], x[1]


def _choice_mask():
    key1 = np.array([0, 1], dtype=np.uint32)
    kc = np.concatenate(_threefry2x32_pair(
        key1, np.zeros(1, np.uint32), np.zeros(1, np.uint32)))
    i = np.arange(_N, dtype=np.uint32)
    b1, b2 = _threefry2x32_pair(kc, np.zeros(_N, np.uint32), i)
    bits = b1 ^ b2
    u = ((bits >> np.uint32(9)) | np.uint32(0x3F800000)).view(np.float32)
    return (u - np.float32(1.0)) < np.float32(_P)


_mask = _choice_mask()
_even_sel = _mask[0::2]
_odd_sel = _mask[1::2]
_pair_pattern = _even_sel.astype(np.int8) + 2 * _odd_sel.astype(np.int8)
_pair_ids = np.arange(_N // 2, dtype=np.int32)

_wid_of_pair = (_pair_ids // _BLK) % _NW
_chunk_plan = []
_idx_rows = []
_per_worker_lists = []
for _pat in (1, 2, 3):
    _ids = _pair_ids[_pair_pattern == _pat]
    _wl = [_ids[_wid_of_pair[_ids] == _w] for _w in range(_NW)]
    assert min(len(x) for x in _wl) > 0
    _nch = -(-max(len(x) for x in _wl) // _CK)
    _chunk_plan.extend([_pat - 1] * _nch)
    _per_worker_lists.append((_wl, _nch))
_NCHUNK = len(_chunk_plan)
assert _NCHUNK <= _NW
_idx_np = np.empty((_NW, _NCHUNK, _CK), dtype=np.int32)
for _w in range(_NW):
    _parts = []
    for _wl, _nch in _per_worker_lists:
        _pad = np.full(_nch * _CK, _wl[_w][0], dtype=np.int32)
        _pad[: _wl[_w].size] = _wl[_w]
        _parts.append(_pad)
    _idx_np[_w] = np.concatenate(_parts).reshape(_NCHUNK, _CK)

_mesh = plsc.VectorSubcoreMesh(core_axis_name="c", subcore_axis_name="s",
                               num_cores=2, num_subcores=16)


@functools.partial(
    pl.kernel,
    out_type=jax.ShapeDtypeStruct((_PAIRS, _PD), jnp.float32),
    mesh=_mesh,
    scratch_types=[
        pltpu.VMEM((_BLK, _PD), jnp.float32),
        pltpu.VMEM((_BLK, _PD), jnp.float32),
        pltpu.VMEM((_CK, _PD), jnp.float32),
        pltpu.VMEM((_CK, _PD), jnp.float32),
        pltpu.VMEM((_NCHUNK, _CK), jnp.int32),
        pltpu.VMEM((4, _PD), jnp.float32),
        pltpu.SemaphoreType.DMA,
        pltpu.SemaphoreType.DMA,
        pltpu.SemaphoreType.DMA,
        pltpu.SemaphoreType.DMA,
        pltpu.SemaphoreType.DMA,
        pltpu.SemaphoreType.DMA,
        pltpu.SemaphoreType.DMA,
        pltpu.SemaphoreType.DMA,
    ],
)
def _sc_noise_kernel(data_h, idx_h, nvec_h, out_h,
                     buf0, buf1, rows0, rows1, idx_v, nvec_v,
                     sin0, sin1, sout0, sout1, fin0, fin1, fout0, fout1):
    w = lax.axis_index("s") * 2 + lax.axis_index("c")
    bufs = (buf0, buf1)
    sins = (sin0, sin1)
    souts = (sout0, sout1)
    rows = (rows0, rows1)
    fins = (fin0, fin1)
    fouts = (fout0, fout1)

    def start(i):
        return (w + i * _NW) * _BLK

    pltpu.sync_copy(nvec_h, nvec_v)
    pltpu.sync_copy(idx_h.at[w], idx_v)

    def fix_gather(c, b):
        pltpu.make_async_copy(data_h.at[idx_v.at[c]], rows[b], fins[b]).start()

    def fix_step(c):
        b = c % 2
        pltpu.make_async_copy(data_h.at[idx_v.at[c]], rows[b], fins[b]).wait()
        pat = _chunk_plan[c]
        carry0 = tuple(nvec_v[pat, pl.ds(q * 16, 16)] for q in range(_PD // 16))

        def add_vec(k, carry):
            for q in range(_PD // 16):
                rows[b][k, pl.ds(q * 16, 16)] += carry[q]
            return carry

        lax.fori_loop(0, _CK, add_vec, carry0)
        sc = pltpu.make_async_copy(rows[b], out_h.at[idx_v.at[c]], fouts[b])
        sc.start()
        if c + 2 < _NCHUNK:
            sc.wait()
            fix_gather(c + 2, b)

    pltpu.make_async_copy(data_h.at[pl.ds(start(0), _BLK)], buf0, sin0).start()
    pltpu.make_async_copy(data_h.at[pl.ds(start(1), _BLK)], buf1, sin1).start()
    half = _NBLK_W // 2
    for i in range(_NBLK_W):
        b = i % 2
        pltpu.make_async_copy(
            data_h.at[pl.ds(start(i), _BLK)], bufs[b], sins[b]).wait()
        sc = pltpu.make_async_copy(
            bufs[b], out_h.at[pl.ds(start(i), _BLK)], souts[b])
        sc.start()
        if i + 2 < _NBLK_W:
            sc.wait()
            pltpu.make_async_copy(
                data_h.at[pl.ds(start(i + 2), _BLK)], bufs[b], sins[b]).start()
        if i == half - 1:
            fix_gather(0, 0)
            fix_gather(1, 1)
        if half <= i < half + _NCHUNK:
            fix_step(i - half)
    for i in (_NBLK_W - 2, _NBLK_W - 1):
        pltpu.make_async_copy(
            bufs[i % 2], out_h.at[pl.ds(start(i), _BLK)], souts[i % 2]).wait()
    for c in (_NCHUNK - 2, _NCHUNK - 1):
        pltpu.make_async_copy(
            rows[c % 2], out_h.at[idx_v.at[c]], fouts[c % 2]).wait()


def kernel(data):
    paired = data.reshape(_PAIRS, _PD)
    noise = _MEAN + _SIGMA * jax.random.normal(
        jax.random.fold_in(jax.random.key(1), 1), (_D,), dtype=jnp.float32)
    zero = jnp.zeros((_D,), jnp.float32)
    nvec = jnp.stack([
        jnp.concatenate([noise, zero]),
        jnp.concatenate([zero, noise]),
        jnp.concatenate([noise, noise]),
        jnp.concatenate([zero, zero]),
    ])
    out = _sc_noise_kernel(paired, jnp.asarray(_idx_np), nvec)
    return out.reshape(data.shape)


# speedup vs baseline: 6.7792x; 6.7792x over previous
import functools
import math

import jax
import jax.numpy as jnp
import numpy as np
from jax import lax
from jax.experimental import pallas as pl
from jax.experimental.pallas import tpu as pltpu
from jax.experimental.pallas import tpu_sc as plsc

_P = 0.1
_MEAN = 0.0
_SIGMA = 0.01
_N = 524288
_D = 64
_NW = 32
_W = 256
_NBLK = _N // (_W * _NW)
_NIT = 2 * _NBLK
_NBUF = 4


def _rotl(x, d):
    return ((x << np.uint32(d)) | (x >> np.uint32(32 - d))).astype(np.uint32)


def _threefry2x32_pair(key, x0, x1):
    x = [x0.astype(np.uint32).copy(), x1.astype(np.uint32).copy()]
    rotations = [(13, 15, 26, 6), (17, 29, 16, 24)]
    ks = [key[0], key[1], np.uint32(key[0] ^ key[1] ^ np.uint32(0x1BD11BDA))]
    x[0] = (x[0] + ks[0]).astype(np.uint32)
    x[1] = (x[1] + ks[1]).astype(np.uint32)
    for i in range(5):
        for r in rotations[i % 2]:
            x[0] = (x[0] + x[1]).astype(np.uint32)
            x[1] = _rotl(x[1], r)
            x[1] = x[1] ^ x[0]
        x[0] = (x[0] + ks[(i + 1) % 3]).astype(np.uint32)
        x[1] = (x[1] + ks[(i + 2) % 3] + np.uint32(i + 1)).astype(np.uint32)
    return x[0], x[1]


def _random_u01(key, n):
    i = np.arange(n, dtype=np.uint32)
    b1, b2 = _threefry2x32_pair(key, np.zeros(n, np.uint32), i)
    bits = b1 ^ b2
    return ((bits >> np.uint32(9)) | np.uint32(0x3F800000)).view(np.float32) \
        - np.float32(1.0)


def _fold_in(key, d):
    return np.concatenate(_threefry2x32_pair(
        key, np.zeros(1, np.uint32), np.full(1, d, np.uint32)))


def _erfinv(y):
    x = 0.0
    for _ in range(60):
        step = (math.erf(x) - y) * (math.sqrt(math.pi) / 2.0) * math.exp(x * x)
        x -= step
        if abs(step) < 1e-17:
            break
    return x


_key1 = np.array([0, 1], dtype=np.uint32)
_mask_np = _random_u01(_fold_in(_key1, 0), _N) < np.float32(_P)

_lo = np.float32(np.nextafter(np.float32(-1), np.float32(0)))
_u = _random_u01(_fold_in(_key1, 1), _D) * (np.float32(1.0) - _lo) + _lo
_u = np.maximum(_lo, _u)
_nrm = np.array([math.sqrt(2.0) * _erfinv(float(v)) for v in _u],
                dtype=np.float32)
_noise_np = (np.float32(_MEAN) + np.float32(_SIGMA) * _nrm).astype(np.float32)
_NOISE = [float(v) for v in _noise_np]

_mesh = plsc.VectorSubcoreMesh(core_axis_name="c", subcore_axis_name="s",
                               num_cores=2, num_subcores=16)


@functools.partial(
    pl.kernel,
    out_type=jax.ShapeDtypeStruct((2, _D, _N), jnp.float32),
    mesh=_mesh,
    scratch_types=[
        [pltpu.VMEM((_D, _W), jnp.float32) for _ in range(_NBUF)],
        pltpu.VMEM((_NBLK, _W), jnp.float32),
        [pltpu.SemaphoreType.DMA for _ in range(_NBUF)],
        [pltpu.SemaphoreType.DMA for _ in range(_NBUF)],
    ],
)
def _sc_noise_kernel(data_h, mask_h, out_h, bufs, mask_v, gsems, ssems):
    w = lax.axis_index("s") * 2 + lax.axis_index("c")

    pltpu.sync_copy(mask_h.at[:, w], mask_v)

    def n_start(j):
        return (w + j * _NW) * _W

    def gather_start(i, b):
        pltpu.make_async_copy(
            data_h.at[b % 2, :, pl.ds(n_start(i // 2), _W)],
            bufs[b % _NBUF], gsems[b % _NBUF]).start()

    def gather_wait(i, b):
        pltpu.make_async_copy(
            data_h.at[b % 2, :, pl.ds(n_start(i // 2), _W)],
            bufs[b % _NBUF], gsems[b % _NBUF]).wait()

    def scatter(i, b):
        return pltpu.make_async_copy(
            bufs[b % _NBUF],
            out_h.at[b % 2, :, pl.ds(n_start(i // 2), _W)], ssems[b % _NBUF])

    def add_noise(b, j):
        def body(q, carry):
            s = q * 16
            m = mask_v[j, pl.ds(s, 16)]
            for c in range(_D):
                bufs[b % _NBUF][c, pl.ds(s, 16)] += m * _NOISE[c]
            return carry
        lax.fori_loop(0, _W // 16, body, 0)

    def step(i, b, do_wait, do_next):
        gather_wait(i, b)
        if b % 2 == 0:
            add_noise(b, i // 2)
        scatter(i, b).start()
        if do_next:
            if do_wait:
                scatter(i - 2, b - 2).wait()
            gather_start(i + 2, b + 2)

    gather_start(0, 0)
    gather_start(1, 1)
    for i in range(_NBUF):
        step(i, i, do_wait=i >= 2, do_next=True)

    def outer(o, carry):
        base = o * _NBUF
        for b in range(_NBUF):
            step(base + b, b, do_wait=True, do_next=True)
        return carry

    lax.fori_loop(1, _NIT // _NBUF - 1, outer, 0)

    for i in range(_NIT - _NBUF, _NIT):
        step(i, i, do_wait=True, do_next=i + 2 < _NIT)
    scatter(_NIT - 2, _NIT - 2).wait()
    scatter(_NIT - 1, _NIT - 1).wait()


def kernel(data):
    dt = jnp.transpose(data, (0, 2, 1))
    out = _sc_noise_kernel(
        dt, jnp.asarray(_mask_np.reshape(_NBLK, _NW, _W), jnp.float32))
    return jnp.transpose(out, (0, 2, 1))


# speedup vs baseline: 7.1130x; 1.0492x over previous
import functools
import math

import jax
import jax.numpy as jnp
import numpy as np
from jax import lax
from jax.experimental import pallas as pl
from jax.experimental.pallas import tpu as pltpu
from jax.experimental.pallas import tpu_sc as plsc

_P = 0.1
_MEAN = 0.0
_SIGMA = 0.01
_N = 524288
_D = 64
_NW = 32
_W = 256
_NBLK = _N // (_W * _NW)
_NIT = 2 * _NBLK
_NBUF = 6
_LOOK = 4


def _rotl(x, d):
    return ((x << np.uint32(d)) | (x >> np.uint32(32 - d))).astype(np.uint32)


def _threefry2x32_pair(key, x0, x1):
    x = [x0.astype(np.uint32).copy(), x1.astype(np.uint32).copy()]
    rotations = [(13, 15, 26, 6), (17, 29, 16, 24)]
    ks = [key[0], key[1], np.uint32(key[0] ^ key[1] ^ np.uint32(0x1BD11BDA))]
    x[0] = (x[0] + ks[0]).astype(np.uint32)
    x[1] = (x[1] + ks[1]).astype(np.uint32)
    for i in range(5):
        for r in rotations[i % 2]:
            x[0] = (x[0] + x[1]).astype(np.uint32)
            x[1] = _rotl(x[1], r)
            x[1] = x[1] ^ x[0]
        x[0] = (x[0] + ks[(i + 1) % 3]).astype(np.uint32)
        x[1] = (x[1] + ks[(i + 2) % 3] + np.uint32(i + 1)).astype(np.uint32)
    return x[0], x[1]


def _random_u01(key, n):
    i = np.arange(n, dtype=np.uint32)
    b1, b2 = _threefry2x32_pair(key, np.zeros(n, np.uint32), i)
    bits = b1 ^ b2
    return ((bits >> np.uint32(9)) | np.uint32(0x3F800000)).view(np.float32) \
        - np.float32(1.0)


def _fold_in(key, d):
    return np.concatenate(_threefry2x32_pair(
        key, np.zeros(1, np.uint32), np.full(1, d, np.uint32)))


def _erfinv(y):
    x = 0.0
    for _ in range(60):
        step = (math.erf(x) - y) * (math.sqrt(math.pi) / 2.0) * math.exp(x * x)
        x -= step
        if abs(step) < 1e-17:
            break
    return x


_key1 = np.array([0, 1], dtype=np.uint32)
_mask_np = _random_u01(_fold_in(_key1, 0), _N) < np.float32(_P)

_lo = np.float32(np.nextafter(np.float32(-1), np.float32(0)))
_u = _random_u01(_fold_in(_key1, 1), _D) * (np.float32(1.0) - _lo) + _lo
_u = np.maximum(_lo, _u)
_nrm = np.array([math.sqrt(2.0) * _erfinv(float(v)) for v in _u],
                dtype=np.float32)
_noise_np = (np.float32(_MEAN) + np.float32(_SIGMA) * _nrm).astype(np.float32)
_NOISE = [float(v) for v in _noise_np]

_mesh = plsc.VectorSubcoreMesh(core_axis_name="c", subcore_axis_name="s",
                               num_cores=2, num_subcores=16)


@functools.partial(
    pl.kernel,
    out_type=jax.ShapeDtypeStruct((2, _D, _N), jnp.float32),
    mesh=_mesh,
    scratch_types=[
        [pltpu.VMEM((_D, _W), jnp.float32) for _ in range(_NBUF)],
        pltpu.VMEM((_NBLK, _W), jnp.float32),
        [pltpu.SemaphoreType.DMA for _ in range(_NBUF)],
        [pltpu.SemaphoreType.DMA for _ in range(_NBUF)],
    ],
)
def _sc_noise_kernel(data_h, mask_h, out_h, bufs, mask_v, gsems, ssems):
    w = lax.axis_index("s") * 2 + lax.axis_index("c")

    def n_start(j):
        return (w + j * _NW) * _W

    def gather_start(i, b):
        pltpu.make_async_copy(
            data_h.at[b % 2, :, pl.ds(n_start(i // 2), _W)],
            bufs[b % _NBUF], gsems[b % _NBUF]).start()

    def gather_wait(i, b):
        pltpu.make_async_copy(
            data_h.at[b % 2, :, pl.ds(n_start(i // 2), _W)],
            bufs[b % _NBUF], gsems[b % _NBUF]).wait()

    def scatter(i, b):
        return pltpu.make_async_copy(
            bufs[b % _NBUF],
            out_h.at[b % 2, :, pl.ds(n_start(i // 2), _W)], ssems[b % _NBUF])

    def add_noise(b, j):
        def body(q, carry):
            s = q * 16
            m = mask_v[j, pl.ds(s, 16)]
            for c in range(_D):
                bufs[b % _NBUF][c, pl.ds(s, 16)] += m * _NOISE[c]
            return carry
        lax.fori_loop(0, _W // 16, body, 0)

    def step(i, b, do_wait, do_next):
        gather_wait(i, b)
        if b % 2 == 0:
            add_noise(b, i // 2)
        scatter(i, b).start()
        if do_next:
            if do_wait:
                scatter(i + _LOOK - _NBUF, b + _LOOK - _NBUF).wait()
            gather_start(i + _LOOK, b + _LOOK)

    for k in range(_LOOK):
        gather_start(k, k)
    pltpu.sync_copy(mask_h.at[:, w], mask_v)
    for i in range(_NBUF):
        step(i, i, do_wait=i >= _NBUF - _LOOK, do_next=True)

    def outer(o, carry):
        base = o * _NBUF
        for b in range(_NBUF):
            step(base + b, b, do_wait=True, do_next=True)
        return carry

    _EP0 = ((_NIT - _NBUF) // _NBUF) * _NBUF
    lax.fori_loop(1, _EP0 // _NBUF, outer, 0)

    for i in range(_EP0, _NIT):
        step(i, i, do_wait=True, do_next=i + _LOOK < _NIT)
    for i in range(_NIT - _NBUF, _NIT):
        scatter(i, i).wait()


def kernel(data):
    dt = jnp.transpose(data, (0, 2, 1))
    out = _sc_noise_kernel(
        dt, jnp.asarray(_mask_np.reshape(_NBLK, _NW, _W), jnp.float32))
    return jnp.transpose(out, (0, 2, 1))
